# Initial kernel scaffold; baseline (speedup 1.0000x reference)
import jax
import jax.numpy as jnp
from jax.experimental import pallas as pl


def kernel(x, edge_index, x_batch, W1l, b1, W1r, W2l, b2, W2r):
    raise NotImplementedError("write your pallas kernel here")


# speedup vs baseline: 4.2602x; 4.2602x over previous
import functools

import jax
import jax.numpy as jnp
from jax import lax
from jax.experimental import pallas as pl
from jax.experimental.pallas import tpu as pltpu
from jax.experimental.pallas import tpu_sc as plsc

N = 10000
D = 128
E = 320000

NC = 2
NS = 16
C = 128
K = 80
KB = 5
SK = K // KB
EPT = K * C
E_PAD = NC * NS * EPT
NP = 10240
RPT = NP // NS


def _sc_agg_build(with_cnt: bool):
    mesh = plsc.VectorSubcoreMesh(core_axis_name="c", subcore_axis_name="s")
    if with_cnt:
        out_type = (jax.ShapeDtypeStruct((NC, NP, D), jnp.float32),
                    jax.ShapeDtypeStruct((NC, NP), jnp.float32))
    else:
        out_type = jax.ShapeDtypeStruct((NC, NP, D), jnp.float32)

    scratch = (
        pltpu.VMEM((SK, C), jnp.int32),
        pltpu.VMEM((SK, C), jnp.int32),
        pltpu.VMEM((C, D), jnp.float32),
        pltpu.VMEM((C, D), jnp.float32),
        pltpu.VMEM((C,), jnp.float32),
        pltpu.VMEM((RPT,), jnp.float32),
        pltpu.VMEM_SHARED((NP, D), jnp.float32),
        pltpu.VMEM_SHARED((NP,), jnp.float32),
        pltpu.SemaphoreType.DMA,
        pltpu.SemaphoreType.DMA,
    )

    def body(x_hbm, src_hbm, dst_hbm, agg_out, *rest):
        if with_cnt:
            cnt_out = rest[0]
            rest = rest[1:]
        else:
            cnt_out = None
        src_v, dst_v, rows0, rows1, ones_v, cntbuf, acc_sh, cnt_sh, sem0, sem1 = rest
        cid = lax.axis_index("c")
        sid = lax.axis_index("s")
        rows = (rows0, rows1)
        sems = (sem0, sem1)

        z16 = jnp.zeros((16,), jnp.float32)

        def zrow(i, _):
            for k in range(D // 16):
                rows0[i, pl.ds(k * 16, 16)] = z16
            return 0

        lax.fori_loop(0, C, zrow, 0)

        def zcnt(i, _):
            cntbuf[pl.ds(i * 16, 16)] = z16
            return 0

        lax.fori_loop(0, RPT // 16, zcnt, 0)
        for k in range(C // 16):
            ones_v[pl.ds(k * 16, 16)] = jnp.ones((16,), jnp.float32)

        base = sid * RPT
        for t in range(RPT // C):
            pltpu.sync_copy(rows0, acc_sh.at[pl.ds(base + t * C, C)])
        if with_cnt:
            pltpu.sync_copy(cntbuf, cnt_sh.at[pl.ds(base, RPT)])
        plsc.subcore_barrier()

        def gather(j, b):
            return pltpu.async_copy(x_hbm.at[src_v.at[j]], rows[b], sems[b])

        def gwait(j, b):
            pltpu.make_async_copy(x_hbm.at[src_v.at[j]], rows[b], sems[b]).wait()

        def scat(j, b):
            pltpu.sync_copy(rows[b], acc_sh.at[dst_v.at[j]], add=True)
            if with_cnt:
                pltpu.sync_copy(ones_v, cnt_sh.at[dst_v.at[j]], add=True)

        def stage(st, _):
            pltpu.sync_copy(src_hbm.at[cid, sid, pl.ds(st * SK, SK)], src_v)
            pltpu.sync_copy(dst_hbm.at[cid, sid, pl.ds(st * SK, SK)], dst_v)
            gather(0, 0)
            gather(1, 1)

            def step(g, _):
                j = 2 * g
                for b in range(2):
                    gwait(j + b, b)
                    scat(j + b, b)
                    gather(j + b + 2, b)
                return 0

            lax.fori_loop(0, SK // 2 - 1, step, 0)
            for b in range(2):
                j = SK - 2 + b
                gwait(j, b)
                scat(j, b)
            return 0

        lax.fori_loop(0, KB, stage, 0)

        plsc.subcore_barrier()
        for t in range(RPT // C):
            pltpu.sync_copy(acc_sh.at[pl.ds(base + t * C, C)], rows0)
            pltpu.sync_copy(rows0, agg_out.at[cid, pl.ds(base + t * C, C)])
        if with_cnt:
            pltpu.sync_copy(cnt_sh.at[pl.ds(base, RPT)], cntbuf)
            pltpu.sync_copy(cntbuf, cnt_out.at[cid, pl.ds(base, RPT)])

    return pl.kernel(body, out_type=out_type, mesh=mesh,
                     scratch_types=scratch)


_sc_agg_cnt = _sc_agg_build(with_cnt=True)
_sc_agg = _sc_agg_build(with_cnt=False)


_RB = 1280


def _tc_body(agg_ref, cnt_ref, x_ref, wl_ref, wr_ref, b_ref, out_ref):
    cnt = cnt_ref[...]
    inv = 1.0 / jnp.maximum(cnt[:, 0:1] + cnt[:, 1:2], 1.0)
    mean = (agg_ref[0] + agg_ref[1]) * inv
    dn = (((1,), (1,)), ((), ()))
    out_ref[...] = (
        lax.dot_general(mean, wl_ref[...], dn, preferred_element_type=jnp.float32)
        + lax.dot_general(x_ref[...], wr_ref[...], dn, preferred_element_type=jnp.float32)
        + b_ref[...]
    )


_tc_combine = pl.pallas_call(
    _tc_body,
    grid=(NP // _RB,),
    in_specs=[
        pl.BlockSpec((NC, _RB, D), lambda i: (0, i, 0)),
        pl.BlockSpec((_RB, NC), lambda i: (i, 0)),
        pl.BlockSpec((_RB, D), lambda i: (i, 0)),
        pl.BlockSpec((D, D), lambda i: (0, 0)),
        pl.BlockSpec((D, D), lambda i: (0, 0)),
        pl.BlockSpec((1, D), lambda i: (0, 0)),
    ],
    out_specs=pl.BlockSpec((_RB, D), lambda i: (i, 0)),
    out_shape=jax.ShapeDtypeStruct((NP, D), jnp.float32),
)


def kernel(x, edge_index, x_batch, W1l, b1, W1r, W2l, b2, W2r):
    src = edge_index[0].astype(jnp.int32)
    dst = edge_index[1].astype(jnp.int32)
    npad = E_PAD - E
    src = jnp.concatenate([src, jnp.zeros((npad,), jnp.int32)])
    dst = jnp.concatenate([dst, N + (jnp.arange(npad, dtype=jnp.int32) % (NP - N))])
    src_r = src.reshape(NC, NS, K, C)
    dst_r = dst.reshape(NC, NS, K, C)
    x_pad = jnp.concatenate([x, jnp.zeros((NP - N, D), jnp.float32)])

    agg1, cnt = _sc_agg_cnt(x_pad, src_r, dst_r)
    cnt_t = cnt.T
    h = _tc_combine(agg1, cnt_t, x_pad, W1l, W1r, b1.reshape(1, D))
    agg2 = _sc_agg(h, src_r, dst_r)
    out = _tc_combine(agg2, cnt_t, h, W2l, W2r, b2.reshape(1, D))
    return out[:N]


# speedup vs baseline: 12.7313x; 2.9884x over previous
import functools

import jax
import jax.numpy as jnp
from jax import lax
from jax.experimental import pallas as pl
from jax.experimental.pallas import tpu as pltpu
from jax.experimental.pallas import tpu_sc as plsc

N = 10000
D = 128
E = 320000

NC = 2
NS = 16
C = 128
K = 80
KB = 5
SK = K // KB
EPT = K * C
E_PAD = NC * NS * EPT
NP = 10240
RPT = NP // NS


def _sc_agg_build(with_cnt: bool):
    mesh = plsc.VectorSubcoreMesh(core_axis_name="c", subcore_axis_name="s")
    if with_cnt:
        out_type = (jax.ShapeDtypeStruct((NC, NP, D), jnp.float32),
                    jax.ShapeDtypeStruct((NC, NP), jnp.float32))
    else:
        out_type = jax.ShapeDtypeStruct((NC, NP, D), jnp.float32)

    scratch = (
        pltpu.VMEM((SK, C), jnp.int32),
        pltpu.VMEM((SK, C), jnp.int32),
        pltpu.VMEM((C, D), jnp.float32),
        pltpu.VMEM((C, D), jnp.float32),
        pltpu.VMEM((C,), jnp.float32),
        pltpu.VMEM((RPT,), jnp.float32),
        pltpu.VMEM_SHARED((NP, D), jnp.float32),
        pltpu.VMEM_SHARED((NP,), jnp.float32),
        pltpu.SemaphoreType.DMA,
        pltpu.SemaphoreType.DMA,
    )

    def body(x_hbm, src_hbm, dst_hbm, agg_out, *rest):
        if with_cnt:
            cnt_out = rest[0]
            rest = rest[1:]
        else:
            cnt_out = None
        src_v, dst_v, rows0, rows1, ones_v, cntbuf, acc_sh, cnt_sh, sem0, sem1 = rest
        cid = lax.axis_index("c")
        sid = lax.axis_index("s")
        rows = (rows0, rows1)
        sems = (sem0, sem1)

        z16 = jnp.zeros((16,), jnp.float32)

        def zrow(i, _):
            for k in range(D // 16):
                rows0[i, pl.ds(k * 16, 16)] = z16
            return 0

        lax.fori_loop(0, C, zrow, 0)

        def zcnt(i, _):
            cntbuf[pl.ds(i * 16, 16)] = z16
            return 0

        lax.fori_loop(0, RPT // 16, zcnt, 0)
        for k in range(C // 16):
            ones_v[pl.ds(k * 16, 16)] = jnp.ones((16,), jnp.float32)

        base = sid * RPT
        for t in range(RPT // C):
            pltpu.sync_copy(rows0, acc_sh.at[pl.ds(base + t * C, C)])
        if with_cnt:
            pltpu.sync_copy(cntbuf, cnt_sh.at[pl.ds(base, RPT)])
        plsc.subcore_barrier()

        def gather(j, b):
            return pltpu.async_copy(x_hbm.at[src_v.at[j]], rows[b], sems[b])

        def gwait(j, b):
            pltpu.make_async_copy(x_hbm.at[src_v.at[j]], rows[b], sems[b]).wait()

        def scat(j, b):
            pltpu.sync_copy(rows[b], acc_sh.at[dst_v.at[j]], add=True)
            if with_cnt:
                pltpu.sync_copy(ones_v, cnt_sh.at[dst_v.at[j]], add=True)

        def stage(st, _):
            pltpu.sync_copy(src_hbm.at[cid, sid, pl.ds(st * SK, SK)], src_v)
            pltpu.sync_copy(dst_hbm.at[cid, sid, pl.ds(st * SK, SK)], dst_v)
            gather(0, 0)
            gather(1, 1)

            def step(g, _):
                j = 2 * g
                for b in range(2):
                    gwait(j + b, b)
                    scat(j + b, b)
                    gather(j + b + 2, b)
                return 0

            lax.fori_loop(0, SK // 2 - 1, step, 0)
            for b in range(2):
                j = SK - 2 + b
                gwait(j, b)
                scat(j, b)
            return 0

        lax.fori_loop(0, KB, stage, 0)

        plsc.subcore_barrier()
        for t in range(RPT // C):
            pltpu.sync_copy(acc_sh.at[pl.ds(base + t * C, C)], rows0)
            pltpu.sync_copy(rows0, agg_out.at[cid, pl.ds(base + t * C, C)])
        if with_cnt:
            pltpu.sync_copy(cnt_sh.at[pl.ds(base, RPT)], cntbuf)
            pltpu.sync_copy(cntbuf, cnt_out.at[cid, pl.ds(base, RPT)])

    return pl.kernel(body, out_type=out_type, mesh=mesh,
                     scratch_types=scratch)


_sc_agg_cnt = _sc_agg_build(with_cnt=True)
_sc_agg = _sc_agg_build(with_cnt=False)


_RB = 1280


def _tc_body(agg_ref, cnt_ref, x_ref, wl_ref, wr_ref, b_ref, out_ref):
    cnt = cnt_ref[...]
    inv = 1.0 / jnp.maximum(cnt[:, 0:1] + cnt[:, 1:2], 1.0)
    mean = (agg_ref[0] + agg_ref[1]) * inv
    dn = (((1,), (1,)), ((), ()))
    out_ref[...] = (
        lax.dot_general(mean, wl_ref[...], dn, preferred_element_type=jnp.float32)
        + lax.dot_general(x_ref[...], wr_ref[...], dn, preferred_element_type=jnp.float32)
        + b_ref[...]
    )


_tc_combine = pl.pallas_call(
    _tc_body,
    grid=(NP // _RB,),
    in_specs=[
        pl.BlockSpec((NC, _RB, D), lambda i: (0, i, 0)),
        pl.BlockSpec((_RB, NC), lambda i: (i, 0)),
        pl.BlockSpec((_RB, D), lambda i: (i, 0)),
        pl.BlockSpec((D, D), lambda i: (0, 0)),
        pl.BlockSpec((D, D), lambda i: (0, 0)),
        pl.BlockSpec((1, D), lambda i: (0, 0)),
    ],
    out_specs=pl.BlockSpec((_RB, D), lambda i: (i, 0)),
    out_shape=jax.ShapeDtypeStruct((NP, D), jnp.float32),
)


def kernel(x, edge_index, x_batch, W1l, b1, W1r, W2l, b2, W2r):
    src = edge_index[0].astype(jnp.int32)
    dst = edge_index[1].astype(jnp.int32)
    npad = E_PAD - E
    src = jnp.concatenate([src, jnp.arange(npad, dtype=jnp.int32) % N])
    dst = jnp.concatenate([dst, N + (jnp.arange(npad, dtype=jnp.int32) % (NP - N))])
    src_r = src.reshape(NC, NS, K, C)
    dst_r = dst.reshape(NC, NS, K, C)
    x_pad = jnp.concatenate([x, jnp.zeros((NP - N, D), jnp.float32)])

    agg1, cnt = _sc_agg_cnt(x_pad, src_r, dst_r)
    cnt_t = cnt.T
    h = _tc_combine(agg1, cnt_t, x_pad, W1l, W1r, b1.reshape(1, D))
    agg2 = _sc_agg(h, src_r, dst_r)
    out = _tc_combine(agg2, cnt_t, h, W2l, W2r, b2.reshape(1, D))
    return out[:N]
